# Initial kernel scaffold; baseline (speedup 1.0000x reference)
import jax
import jax.numpy as jnp
from jax.experimental import pallas as pl


def kernel(X_n, edge_index, edge_attr, PE, params):
    raise NotImplementedError("write your pallas kernel here")


# speedup vs baseline: 1.0088x; 1.0088x over previous
import functools
import jax
import jax.numpy as jnp
from jax.experimental import pallas as pl
from jax.experimental.pallas import tpu as pltpu

N_BLK = 1000


def _proj_body(x_ref, w_ref, b_ref, out_ref):
    out_ref[...] = (
        jnp.dot(x_ref[...], w_ref[...], preferred_element_type=jnp.float32)
        + b_ref[...]
    )


def _node_proj(x, wcat, bcat):
    n = x.shape[0]
    grid = n // N_BLK
    return pl.pallas_call(
        _proj_body,
        grid=(grid,),
        in_specs=[
            pl.BlockSpec((N_BLK, 128), lambda i: (i, 0)),
            pl.BlockSpec((128, wcat.shape[1]), lambda i: (0, 0)),
            pl.BlockSpec((1, wcat.shape[1]), lambda i: (0, 0)),
        ],
        out_specs=pl.BlockSpec((N_BLK, wcat.shape[1]), lambda i: (i, 0)),
        out_shape=jax.ShapeDtypeStruct((n, wcat.shape[1]), jnp.float32),
    )(x, wcat, bcat)


def _bn(x, gamma, beta, eps=1e-5):
    mu = jnp.mean(x, axis=0)
    var = jnp.mean((x - mu) ** 2, axis=0)
    return gamma * (x - mu) / jnp.sqrt(var + eps) + beta


def kernel(X_n, edge_index, edge_attr, PE, params):
    src = edge_index[0]
    dst = edge_index[1]
    tables = params["bond_tables"]
    e = tables[0][edge_attr[:, 0]] + tables[1][edge_attr[:, 1]] + tables[2][edge_attr[:, 2]]
    x = X_n
    for lp in params["layers"]:
        x_in = x
        e_in = e
        wcat = jnp.concatenate([lp["WA"], lp["WB"], lp["WD"], lp["WE"]], axis=1)
        bcat = jnp.concatenate([lp["bA"], lp["bB"], lp["bD"], lp["bE"]])[None, :]
        proj = _node_proj(x, wcat, bcat)
        Ax, Bx, Dx, Ex = jnp.split(proj, 4, axis=1)
        Ce = e @ lp["WC"] + lp["bC"]
        e_ij = Dx[dst] + Ex[src] + Ce
        sigma_ij = jax.nn.sigmoid(e_ij)
        num = jax.ops.segment_sum(sigma_ij * Bx[src], dst, num_segments=x.shape[0])
        den = jax.ops.segment_sum(sigma_ij, dst, num_segments=x.shape[0])
        aggr = num / (den + 1e-6)
        x_new = Ax + aggr
        e_new = e_ij
        x_new = _bn(x_new, lp["gamma_x"], lp["beta_x"])
        e_new = _bn(e_new, lp["gamma_e"], lp["beta_e"])
        x_new = jax.nn.gelu(x_new, approximate=False)
        e_new = jax.nn.gelu(e_new, approximate=False)
        x = x_in + x_new
        e = e_in + e_new
    return x


# speedup vs baseline: 2.5093x; 2.4874x over previous
import functools
import jax
import jax.numpy as jnp
from jax import lax
from jax.experimental import pallas as pl
from jax.experimental.pallas import tpu as pltpu
from jax.experimental.pallas import tpu_sc as plsc

N = 10000
E = 320000
D = 128
H = 64
NT = 16
EPT = E // NT
CH = 80
NCHUNK = EPT // CH
NPAD = 10240
RPT = NPAD // NT
RCH = 128
F32 = jnp.float32


def _sc_body(want_e, src_h, dst_h, bx_h, dx_h, ex_h, ce_h, *rest):
    if want_e:
        eij_h, num_h, den_h, stats_h = rest[:4]
        scr = rest[4:]
    else:
        num_h, den_h = rest[:2]
        scr = rest[2:]
    (srcv, dstv, srcav, dstav, bxv, dxv, exv, cev, eijv, msgv, sigv,
     zb, statsv, num_s, den_s, gsem) = scr
    bounce = zb

    c = lax.axis_index("c")
    s = lax.axis_index("s")
    cbase = c * N

    def zrow(r, carry):
        for j in range(4):
            zb[r, pl.ds(j * 16, 16)] = jnp.zeros((16,), F32)
        return carry

    lax.fori_loop(0, RCH, zrow, 0)
    for k in range(RPT // RCH):
        r0 = s * RPT + k * RCH
        pltpu.sync_copy(zb, num_s.at[pl.ds(r0, RCH)])
        pltpu.sync_copy(zb, den_s.at[pl.ds(r0, RCH)])
    plsc.subcore_barrier()

    base = s * EPT

    def row(r, rc):
        out = rc
        if want_e:
            sums = list(rc[:4])
            sqs = list(rc[4:])
        for j in range(4):
            sl = pl.ds(j * 16, 16)
            eij = dxv[r, sl] + exv[r, sl] + cev[r, sl]
            sig = 1.0 / (1.0 + jnp.exp(-eij))
            msgv[r, sl] = sig * bxv[r, sl]
            sigv[r, sl] = sig
            if want_e:
                eijv[r, sl] = eij
                sums[j] = sums[j] + eij
                sqs[j] = sqs[j] + eij * eij
        if want_e:
            out = tuple(sums) + tuple(sqs)
        return out

    def chunk(i, carry):
        off = base + i * CH
        pltpu.sync_copy(src_h.at[pl.ds(off, CH)], srcv)
        pltpu.sync_copy(dst_h.at[pl.ds(off, CH)], dstv)
        for g in range(CH // 16):
            sl = pl.ds(g * 16, 16)
            srcav[sl] = srcv[sl] + cbase
            dstav[sl] = dstv[sl] + cbase
        d1 = pltpu.async_copy(bx_h.at[srcav], bxv, gsem)
        d2 = pltpu.async_copy(ex_h.at[srcav], exv, gsem)
        d3 = pltpu.async_copy(dx_h.at[dstav], dxv, gsem)
        pltpu.sync_copy(ce_h.at[c, pl.ds(off, CH)], cev)
        d1.wait()
        d2.wait()
        d3.wait()
        carry = lax.fori_loop(0, CH, row, carry)
        if want_e:
            pltpu.sync_copy(eijv, eij_h.at[c, pl.ds(off, CH)])
        pltpu.sync_copy(msgv, num_s.at[dstv], add=True)
        pltpu.sync_copy(sigv, den_s.at[dstv], add=True)
        return carry

    if want_e:
        init = tuple(jnp.zeros((16,), F32) for _ in range(8))
    else:
        init = 0
    fin = lax.fori_loop(0, NCHUNK, chunk, init)

    if want_e:
        for j in range(4):
            statsv[0, pl.ds(j * 16, 16)] = fin[j]
            statsv[1, pl.ds(j * 16, 16)] = fin[4 + j]
        pltpu.sync_copy(statsv, stats_h.at[c, s])

    plsc.subcore_barrier()
    for k in range(RPT // RCH):
        r0 = s * RPT + k * RCH
        pltpu.sync_copy(num_s.at[pl.ds(r0, RCH)], bounce)
        pltpu.sync_copy(bounce, num_h.at[c, pl.ds(r0, RCH)])
        pltpu.sync_copy(den_s.at[pl.ds(r0, RCH)], bounce)
        pltpu.sync_copy(bounce, den_h.at[c, pl.ds(r0, RCH)])


def _make_sc_kernel(want_e):
    outs = []
    if want_e:
        outs.append(jax.ShapeDtypeStruct((2, E, H), F32))
    outs.append(jax.ShapeDtypeStruct((2, NPAD, H), F32))
    outs.append(jax.ShapeDtypeStruct((2, NPAD, H), F32))
    if want_e:
        outs.append(jax.ShapeDtypeStruct((2, NT, 2, H), F32))
    scratch = [
        pltpu.VMEM((CH,), jnp.int32),
        pltpu.VMEM((CH,), jnp.int32),
        pltpu.VMEM((CH,), jnp.int32),
        pltpu.VMEM((CH,), jnp.int32),
        pltpu.VMEM((CH, H), F32),
        pltpu.VMEM((CH, H), F32),
        pltpu.VMEM((CH, H), F32),
        pltpu.VMEM((CH, H), F32),
        pltpu.VMEM((CH, H), F32),
        pltpu.VMEM((CH, H), F32),
        pltpu.VMEM((CH, H), F32),
        pltpu.VMEM((RCH, H), F32),
        pltpu.VMEM((2, H), F32),
        pltpu.VMEM_SHARED((NPAD, H), F32),
        pltpu.VMEM_SHARED((NPAD, H), F32),
        pltpu.SemaphoreType.DMA,
    ]
    mesh = plsc.VectorSubcoreMesh(core_axis_name="c", subcore_axis_name="s")
    return pl.kernel(
        functools.partial(_sc_body, want_e),
        out_type=tuple(outs),
        mesh=mesh,
        scratch_types=scratch,
        compiler_params=pltpu.CompilerParams(use_tc_tiling_on_sc=False),
    )


NBLK = 1000
EBLK = 2000


def _gelu(x):
    return 0.5 * x * (1.0 + lax.erf(x * 0.7071067811865476))


def _proj_body(x_ref, w_ref, b_ref, ax_ref, bx_ref, dx_ref, ex_ref):
    p = jnp.dot(x_ref[...], w_ref[...], preferred_element_type=F32) + b_ref[...]
    ax_ref[...] = p[:, :D]
    for k, ref in ((1, bx_ref), (2, dx_ref), (3, ex_ref)):
        half = p[:, k * D:(k + 1) * D]
        ref[...] = jnp.stack([half[:, :H], half[:, H:]], axis=0)


def _node_proj(x, wcat, bcat):
    grid = N // NBLK
    return pl.pallas_call(
        _proj_body,
        grid=(grid,),
        in_specs=[
            pl.BlockSpec((NBLK, D), lambda i: (i, 0)),
            pl.BlockSpec((D, 4 * D), lambda i: (0, 0)),
            pl.BlockSpec((1, 4 * D), lambda i: (0, 0)),
        ],
        out_specs=[
            pl.BlockSpec((NBLK, D), lambda i: (i, 0)),
            pl.BlockSpec((2, NBLK, H), lambda i: (0, i, 0)),
            pl.BlockSpec((2, NBLK, H), lambda i: (0, i, 0)),
            pl.BlockSpec((2, NBLK, H), lambda i: (0, i, 0)),
        ],
        out_shape=[
            jax.ShapeDtypeStruct((N, D), F32),
            jax.ShapeDtypeStruct((2, N, H), F32),
            jax.ShapeDtypeStruct((2, N, H), F32),
            jax.ShapeDtypeStruct((2, N, H), F32),
        ],
    )(x, wcat, bcat)


def _bond_ce_body(attr_ref, tbl_ref, wc_ref, bc_ref, e0_ref, ce_ref):
    attr = attr_ref[...]
    ohs = []
    for k in range(3):
        iota = lax.broadcasted_iota(jnp.int32, (1, 5), 1)
        ohs.append((attr[:, k:k + 1] == iota).astype(F32))
    oh = jnp.concatenate(ohs, axis=1)
    e0 = jnp.dot(oh, tbl_ref[...], preferred_element_type=F32)
    ce = jnp.dot(e0, wc_ref[...], preferred_element_type=F32) + bc_ref[...]
    e0_ref[...] = jnp.stack([e0[:, :H], e0[:, H:]], axis=0)
    ce_ref[...] = jnp.stack([ce[:, :H], ce[:, H:]], axis=0)


def _bond_ce(edge_attr, tblcat, wc, bc):
    grid = E // EBLK
    return pl.pallas_call(
        _bond_ce_body,
        grid=(grid,),
        in_specs=[
            pl.BlockSpec((EBLK, 3), lambda i: (i, 0)),
            pl.BlockSpec((15, D), lambda i: (0, 0)),
            pl.BlockSpec((D, D), lambda i: (0, 0)),
            pl.BlockSpec((1, D), lambda i: (0, 0)),
        ],
        out_specs=[
            pl.BlockSpec((2, EBLK, H), lambda i: (0, i, 0)),
            pl.BlockSpec((2, EBLK, H), lambda i: (0, i, 0)),
        ],
        out_shape=[
            jax.ShapeDtypeStruct((2, E, H), F32),
            jax.ShapeDtypeStruct((2, E, H), F32),
        ],
    )(edge_attr, tblcat, wc, bc)


def _edge_update_ce_body(eij_ref, ep_ref, st_ref, g_ref, b_ref, wc_ref,
                         bc_ref, e_ref, ce_ref):
    st = st_ref[...]
    sums = jnp.sum(st[:, :, 0, :], axis=1)
    sqs = jnp.sum(st[:, :, 1, :], axis=1)
    mu = jnp.concatenate([sums[0], sums[1]])[None, :] * (1.0 / E)
    var = jnp.concatenate([sqs[0], sqs[1]])[None, :] * (1.0 / E) - mu * mu
    inv = lax.rsqrt(var + 1e-5)
    eij = jnp.concatenate([eij_ref[0], eij_ref[1]], axis=1)
    xn = (eij - mu) * inv * g_ref[...] + b_ref[...]
    xn = _gelu(xn)
    enew = jnp.concatenate([ep_ref[0], ep_ref[1]], axis=1) + xn
    ce = jnp.dot(enew, wc_ref[...], preferred_element_type=F32) + bc_ref[...]
    e_ref[...] = jnp.stack([enew[:, :H], enew[:, H:]], axis=0)
    ce_ref[...] = jnp.stack([ce[:, :H], ce[:, H:]], axis=0)


def _edge_update_ce(eij, e_prev, stats, gamma, beta, wc, bc):
    grid = E // EBLK
    return pl.pallas_call(
        _edge_update_ce_body,
        grid=(grid,),
        in_specs=[
            pl.BlockSpec((2, EBLK, H), lambda i: (0, i, 0)),
            pl.BlockSpec((2, EBLK, H), lambda i: (0, i, 0)),
            pl.BlockSpec((2, NT, 2, H), lambda i: (0, 0, 0, 0)),
            pl.BlockSpec((1, D), lambda i: (0, 0)),
            pl.BlockSpec((1, D), lambda i: (0, 0)),
            pl.BlockSpec((D, D), lambda i: (0, 0)),
            pl.BlockSpec((1, D), lambda i: (0, 0)),
        ],
        out_specs=[
            pl.BlockSpec((2, EBLK, H), lambda i: (0, i, 0)),
            pl.BlockSpec((2, EBLK, H), lambda i: (0, i, 0)),
        ],
        out_shape=[
            jax.ShapeDtypeStruct((2, E, H), F32),
            jax.ShapeDtypeStruct((2, E, H), F32),
        ],
    )(eij, e_prev, stats, gamma, beta, wc, bc)


def _node_update_body(ax_ref, num_ref, den_ref, xin_ref, g_ref, b_ref,
                      out_ref):
    num0 = num_ref[0, :N, :]
    num1 = num_ref[1, :N, :]
    den0 = den_ref[0, :N, :]
    den1 = den_ref[1, :N, :]
    aggr = jnp.concatenate(
        [num0 / (den0 + 1e-6), num1 / (den1 + 1e-6)], axis=1)
    xn = ax_ref[...] + aggr
    mu = jnp.mean(xn, axis=0, keepdims=True)
    var = jnp.mean((xn - mu) ** 2, axis=0, keepdims=True)
    xn = (xn - mu) * lax.rsqrt(var + 1e-5) * g_ref[...] + b_ref[...]
    out_ref[...] = xin_ref[...] + _gelu(xn)


def _node_update(ax, num, den, x_in, gamma, beta):
    return pl.pallas_call(
        _node_update_body,
        out_shape=jax.ShapeDtypeStruct((N, D), F32),
    )(ax, num, den, x_in, gamma, beta)


def kernel(X_n, edge_index, edge_attr, PE, params):
    src = edge_index[0]
    dst = edge_index[1]
    tblcat = params["bond_tables"].reshape(3 * 5, D)
    layers = params["layers"]

    es, ce = _bond_ce(edge_attr, tblcat, layers[0]["WC"],
                      layers[0]["bC"][None, :])
    x = X_n
    for l, lp in enumerate(layers):
        wcat = jnp.concatenate([lp["WA"], lp["WB"], lp["WD"], lp["WE"]],
                               axis=1)
        bcat = jnp.concatenate([lp["bA"], lp["bB"], lp["bD"], lp["bE"]])[None, :]
        ax, bx3, dx3, ex3 = _node_proj(x, wcat, bcat)
        bx2 = bx3.reshape(2 * N, H)
        dx2 = dx3.reshape(2 * N, H)
        ex2 = ex3.reshape(2 * N, H)
        want_e = l + 1 < len(layers)
        sc = _make_sc_kernel(want_e)
        if want_e:
            eijs, num, den, stats = sc(src, dst, bx2, dx2, ex2, ce)
        else:
            num, den = sc(src, dst, bx2, dx2, ex2, ce)
        x = _node_update(ax, num, den, x, lp["gamma_x"][None, :],
                         lp["beta_x"][None, :])
        if want_e:
            nlp = layers[l + 1]
            es, ce = _edge_update_ce(eijs, es, stats, lp["gamma_e"][None, :],
                                     lp["beta_e"][None, :], nlp["WC"],
                                     nlp["bC"][None, :])
    return x


# speedup vs baseline: 3.6896x; 1.4704x over previous
import functools
import jax
import jax.numpy as jnp
from jax import lax
from jax.experimental import pallas as pl
from jax.experimental.pallas import tpu as pltpu
from jax.experimental.pallas import tpu_sc as plsc

N = 10000
E = 320000
D = 128
H = 64
NT = 16
EPT = E // NT
CH = 80
NCHUNK = EPT // CH
NPAD = 10240
RPT = NPAD // NT
RCH = 128
F32 = jnp.float32


def _copy_half(hbm_ref, off, n, vbuf, c, to_hbm=False):
    @pl.when(c == 0)
    def _():
        sl = hbm_ref.at[pl.ds(off, n), pl.ds(0, H)]
        if to_hbm:
            pltpu.sync_copy(vbuf, sl)
        else:
            pltpu.sync_copy(sl, vbuf)

    @pl.when(c == 1)
    def _():
        sl = hbm_ref.at[pl.ds(off, n), pl.ds(H, H)]
        if to_hbm:
            pltpu.sync_copy(vbuf, sl)
        else:
            pltpu.sync_copy(sl, vbuf)


def _sc_body(want_e, src_h, dst_h, bx_h, dx_h, ex_h, ce_h, *rest):
    if want_e:
        eij_h, num_h, den_h, stats_h = rest[:4]
        scr = rest[4:]
    else:
        num_h, den_h = rest[:2]
        scr = rest[2:]
    (srcv, dstv, bxv, dxv, exv, cev, eijv, msgv, sigv,
     zb, statsv, num_s, den_s, gsem) = scr
    bounce = zb

    c = lax.axis_index("c")
    s = lax.axis_index("s")

    def zrow(r, carry):
        for j in range(4):
            zb[r, pl.ds(j * 16, 16)] = jnp.zeros((16,), F32)
        return carry

    lax.fori_loop(0, RCH, zrow, 0)
    for k in range(RPT // RCH):
        r0 = s * RPT + k * RCH
        pltpu.sync_copy(zb, num_s.at[pl.ds(r0, RCH)])
        pltpu.sync_copy(zb, den_s.at[pl.ds(r0, RCH)])
    plsc.subcore_barrier()

    base = s * EPT
    bxc = bx_h.at[c]
    dxc = dx_h.at[c]
    exc = ex_h.at[c]

    def row(r, rc):
        out = rc
        if want_e:
            sums = list(rc[:4])
            sqs = list(rc[4:])
        for j in range(4):
            sl = pl.ds(j * 16, 16)
            eij = dxv[r, sl] + exv[r, sl] + cev[r, sl]
            sig = 1.0 / (1.0 + jnp.exp(-eij))
            msgv[r, sl] = sig * bxv[r, sl]
            sigv[r, sl] = sig
            if want_e:
                eijv[r, sl] = eij
                sums[j] = sums[j] + eij
                sqs[j] = sqs[j] + eij * eij
        if want_e:
            out = tuple(sums) + tuple(sqs)
        return out

    def chunk(i, carry):
        off = base + i * CH
        pltpu.sync_copy(src_h.at[pl.ds(off, CH)], srcv)
        pltpu.sync_copy(dst_h.at[pl.ds(off, CH)], dstv)
        d1 = pltpu.async_copy(bxc.at[srcv], bxv, gsem)
        d2 = pltpu.async_copy(exc.at[srcv], exv, gsem)
        d3 = pltpu.async_copy(dxc.at[dstv], dxv, gsem)
        _copy_half(ce_h, off, CH, cev, c)
        d1.wait()
        d2.wait()
        d3.wait()
        carry = lax.fori_loop(0, CH, row, carry)
        if want_e:
            _copy_half(eij_h, off, CH, eijv, c, to_hbm=True)
        pltpu.sync_copy(msgv, num_s.at[dstv], add=True)
        pltpu.sync_copy(sigv, den_s.at[dstv], add=True)
        return carry

    if want_e:
        init = tuple(jnp.zeros((16,), F32) for _ in range(8))
    else:
        init = 0
    fin = lax.fori_loop(0, NCHUNK, chunk, init)

    if want_e:
        for j in range(4):
            statsv[0, pl.ds(j * 16, 16)] = fin[j]
            statsv[1, pl.ds(j * 16, 16)] = fin[4 + j]
        pltpu.sync_copy(statsv, stats_h.at[c, s])

    plsc.subcore_barrier()
    for k in range(RPT // RCH):
        r0 = s * RPT + k * RCH
        pltpu.sync_copy(num_s.at[pl.ds(r0, RCH)], bounce)
        pltpu.sync_copy(bounce, num_h.at[c, pl.ds(r0, RCH)])
        pltpu.sync_copy(den_s.at[pl.ds(r0, RCH)], bounce)
        pltpu.sync_copy(bounce, den_h.at[c, pl.ds(r0, RCH)])


def _make_sc_kernel(want_e):
    outs = []
    if want_e:
        outs.append(jax.ShapeDtypeStruct((E, D), F32))
    outs.append(jax.ShapeDtypeStruct((2, NPAD, H), F32))
    outs.append(jax.ShapeDtypeStruct((2, NPAD, H), F32))
    if want_e:
        outs.append(jax.ShapeDtypeStruct((2, NT, 2, H), F32))
    scratch = [
        pltpu.VMEM((CH,), jnp.int32),
        pltpu.VMEM((CH,), jnp.int32),
        pltpu.VMEM((CH, H), F32),
        pltpu.VMEM((CH, H), F32),
        pltpu.VMEM((CH, H), F32),
        pltpu.VMEM((CH, H), F32),
        pltpu.VMEM((CH, H), F32),
        pltpu.VMEM((CH, H), F32),
        pltpu.VMEM((CH, H), F32),
        pltpu.VMEM((RCH, H), F32),
        pltpu.VMEM((2, H), F32),
        pltpu.VMEM_SHARED((NPAD, H), F32),
        pltpu.VMEM_SHARED((NPAD, H), F32),
        pltpu.SemaphoreType.DMA,
    ]
    mesh = plsc.VectorSubcoreMesh(core_axis_name="c", subcore_axis_name="s")
    return pl.kernel(
        functools.partial(_sc_body, want_e),
        out_type=tuple(outs),
        mesh=mesh,
        scratch_types=scratch,
        compiler_params=pltpu.CompilerParams(use_tc_tiling_on_sc=False),
    )


NBLK = 1000
EBLK = 4000


def _gelu(x):
    return 0.5 * x * (1.0 + lax.erf(x * 0.7071067811865476))


def _proj_body(x_ref, w_ref, b_ref, ax_ref, bx_ref, dx_ref, ex_ref):
    p = jnp.dot(x_ref[...], w_ref[...], preferred_element_type=F32) + b_ref[...]
    ax_ref[...] = p[:, :D]
    for k, ref in ((1, bx_ref), (2, dx_ref), (3, ex_ref)):
        half = p[:, k * D:(k + 1) * D]
        ref[...] = jnp.stack([half[:, :H], half[:, H:]], axis=0)


def _node_proj(x, wcat, bcat):
    grid = N // NBLK
    return pl.pallas_call(
        _proj_body,
        grid=(grid,),
        in_specs=[
            pl.BlockSpec((NBLK, D), lambda i: (i, 0)),
            pl.BlockSpec((D, 4 * D), lambda i: (0, 0)),
            pl.BlockSpec((1, 4 * D), lambda i: (0, 0)),
        ],
        out_specs=[
            pl.BlockSpec((NBLK, D), lambda i: (i, 0)),
            pl.BlockSpec((2, NBLK, H), lambda i: (0, i, 0)),
            pl.BlockSpec((2, NBLK, H), lambda i: (0, i, 0)),
            pl.BlockSpec((2, NBLK, H), lambda i: (0, i, 0)),
        ],
        out_shape=[
            jax.ShapeDtypeStruct((N, D), F32),
            jax.ShapeDtypeStruct((2, N, H), F32),
            jax.ShapeDtypeStruct((2, N, H), F32),
            jax.ShapeDtypeStruct((2, N, H), F32),
        ],
    )(x, wcat, bcat)


def _bond_ce_body(attr_ref, tbl_ref, wc_ref, bc_ref, e0_ref, ce_ref):
    attr = attr_ref[...]
    ohs = []
    for k in range(3):
        iota = lax.broadcasted_iota(jnp.int32, (1, 5), 1)
        ohs.append((attr[:, k:k + 1] == iota).astype(F32))
    oh = jnp.concatenate(ohs, axis=1)
    e0 = jnp.dot(oh, tbl_ref[...], preferred_element_type=F32)
    e0_ref[...] = e0
    ce_ref[...] = jnp.dot(e0, wc_ref[...], preferred_element_type=F32) + bc_ref[...]


def _bond_ce(edge_attr, tblcat, wc, bc):
    grid = E // EBLK
    return pl.pallas_call(
        _bond_ce_body,
        grid=(grid,),
        in_specs=[
            pl.BlockSpec((EBLK, 3), lambda i: (i, 0)),
            pl.BlockSpec((15, D), lambda i: (0, 0)),
            pl.BlockSpec((D, D), lambda i: (0, 0)),
            pl.BlockSpec((1, D), lambda i: (0, 0)),
        ],
        out_specs=[
            pl.BlockSpec((EBLK, D), lambda i: (i, 0)),
            pl.BlockSpec((EBLK, D), lambda i: (i, 0)),
        ],
        out_shape=[
            jax.ShapeDtypeStruct((E, D), F32),
            jax.ShapeDtypeStruct((E, D), F32),
        ],
    )(edge_attr, tblcat, wc, bc)


def _edge_update_ce_body(eij_ref, ep_ref, st_ref, g_ref, b_ref, wc_ref,
                         bc_ref, e_ref, ce_ref):
    st = st_ref[...]
    sums = jnp.sum(st[:, :, 0, :], axis=1)
    sqs = jnp.sum(st[:, :, 1, :], axis=1)
    mu = jnp.concatenate([sums[0], sums[1]])[None, :] * (1.0 / E)
    var = jnp.concatenate([sqs[0], sqs[1]])[None, :] * (1.0 / E) - mu * mu
    inv = lax.rsqrt(var + 1e-5)
    xn = (eij_ref[...] - mu) * inv * g_ref[...] + b_ref[...]
    xn = _gelu(xn)
    enew = ep_ref[...] + xn
    e_ref[...] = enew
    ce_ref[...] = jnp.dot(enew, wc_ref[...], preferred_element_type=F32) + bc_ref[...]


def _edge_update_ce(eij, e_prev, stats, gamma, beta, wc, bc):
    grid = E // EBLK
    return pl.pallas_call(
        _edge_update_ce_body,
        grid=(grid,),
        in_specs=[
            pl.BlockSpec((EBLK, D), lambda i: (i, 0)),
            pl.BlockSpec((EBLK, D), lambda i: (i, 0)),
            pl.BlockSpec((2, NT, 2, H), lambda i: (0, 0, 0, 0)),
            pl.BlockSpec((1, D), lambda i: (0, 0)),
            pl.BlockSpec((1, D), lambda i: (0, 0)),
            pl.BlockSpec((D, D), lambda i: (0, 0)),
            pl.BlockSpec((1, D), lambda i: (0, 0)),
        ],
        out_specs=[
            pl.BlockSpec((EBLK, D), lambda i: (i, 0)),
            pl.BlockSpec((EBLK, D), lambda i: (i, 0)),
        ],
        out_shape=[
            jax.ShapeDtypeStruct((E, D), F32),
            jax.ShapeDtypeStruct((E, D), F32),
        ],
    )(eij, e_prev, stats, gamma, beta, wc, bc)


def _node_update_body(ax_ref, num_ref, den_ref, xin_ref, g_ref, b_ref,
                      out_ref):
    num0 = num_ref[0, :N, :]
    num1 = num_ref[1, :N, :]
    den0 = den_ref[0, :N, :]
    den1 = den_ref[1, :N, :]
    aggr = jnp.concatenate(
        [num0 / (den0 + 1e-6), num1 / (den1 + 1e-6)], axis=1)
    xn = ax_ref[...] + aggr
    mu = jnp.mean(xn, axis=0, keepdims=True)
    var = jnp.mean((xn - mu) ** 2, axis=0, keepdims=True)
    xn = (xn - mu) * lax.rsqrt(var + 1e-5) * g_ref[...] + b_ref[...]
    out_ref[...] = xin_ref[...] + _gelu(xn)


def _node_update(ax, num, den, x_in, gamma, beta):
    return pl.pallas_call(
        _node_update_body,
        out_shape=jax.ShapeDtypeStruct((N, D), F32),
    )(ax, num, den, x_in, gamma, beta)


def kernel(X_n, edge_index, edge_attr, PE, params):
    src = edge_index[0]
    dst = edge_index[1]
    tblcat = params["bond_tables"].reshape(3 * 5, D)
    layers = params["layers"]

    es, ce = _bond_ce(edge_attr, tblcat, layers[0]["WC"],
                      layers[0]["bC"][None, :])
    x = X_n
    for l, lp in enumerate(layers):
        wcat = jnp.concatenate([lp["WA"], lp["WB"], lp["WD"], lp["WE"]],
                               axis=1)
        bcat = jnp.concatenate([lp["bA"], lp["bB"], lp["bD"], lp["bE"]])[None, :]
        ax, bx3, dx3, ex3 = _node_proj(x, wcat, bcat)
        want_e = l + 1 < len(layers)
        sc = _make_sc_kernel(want_e)
        if want_e:
            eijs, num, den, stats = sc(src, dst, bx3, dx3, ex3, ce)
        else:
            num, den = sc(src, dst, bx3, dx3, ex3, ce)
        x = _node_update(ax, num, den, x, lp["gamma_x"][None, :],
                         lp["beta_x"][None, :])
        if want_e:
            nlp = layers[l + 1]
            es, ce = _edge_update_ce(eijs, es, stats, lp["gamma_e"][None, :],
                                     lp["beta_e"][None, :], nlp["WC"],
                                     nlp["bC"][None, :])
    return x
